# baseline (speedup 1.0000x reference)
import functools

import jax
import jax.numpy as jnp
from jax import lax
from jax.experimental import pallas as pl
from jax.experimental.pallas import tpu as pltpu
from jax.experimental.pallas import tpu_sc as plsc

N = 10000
NP = 10240
E = 320000
D = 128

NW = 32
C = 128
NCHUNK = E // C
ITERS_A = (NCHUNK + NW - 1) // NW

ZBLK = 80
NBLK = N // ZBLK

R2 = 640
EH = E // 2
CS = 2000
NSCAN = EH // CS
GROUPS = CS // 16
B = 96

_mesh = plsc.VectorSubcoreMesh(core_axis_name="c", subcore_axis_name="s")
_sc_params = pltpu.CompilerParams(needs_layout_passes=False)


@functools.partial(
    pl.kernel,
    mesh=_mesh,
    out_type=[
        jax.ShapeDtypeStruct((2, N, D), jnp.float32),
        jax.ShapeDtypeStruct((2 * NP,), jnp.float32),
    ],
    compiler_params=_sc_params,
    scratch_types=[
        pltpu.VMEM((C,), jnp.int32),
        pltpu.VMEM((C,), jnp.int32),
        pltpu.VMEM((C, D), jnp.float32),
        pltpu.VMEM((C,), jnp.int32),
        pltpu.VMEM((C,), jnp.int32),
        pltpu.VMEM((C, D), jnp.float32),
        pltpu.VMEM((C,), jnp.float32),
        pltpu.VMEM((2048,), jnp.float32),
        pltpu.VMEM_SHARED((N, D), jnp.float32),
        pltpu.VMEM_SHARED((NP,), jnp.float32),
        pltpu.SemaphoreType.DMA,
        pltpu.SemaphoreType.DMA,
        pltpu.SemaphoreType.DMA,
        pltpu.SemaphoreType.DMA,
    ],
)
def _sc_gather_add(x_hbm, src_hbm, dst_hbm, z128_hbm, z2k_hbm, ones_hbm,
                   agg_hbm, cnt_hbm,
                   srcv0, dstv0, rows0, srcv1, dstv1, rows1,
                   onesv, cstage, agg_sh, cnt_sh, sem0, sem1, sems0, sems1):
    c = lax.axis_index("c")
    s = lax.axis_index("s")
    tid = s * 2 + c

    pltpu.sync_copy(z128_hbm, rows0.at[pl.ds(0, ZBLK)])
    pltpu.sync_copy(ones_hbm, onesv)
    pltpu.sync_copy(z2k_hbm, cstage)
    for b in range(8):
        bid = b * 16 + s

        @pl.when(bid < NBLK)
        def _():
            pltpu.sync_copy(rows0.at[pl.ds(0, ZBLK)],
                            agg_sh.at[pl.ds(bid * ZBLK, ZBLK)])

    @pl.when(s < 5)
    def _():
        pltpu.sync_copy(cstage, cnt_sh.at[pl.ds(s * 2048, 2048)])

    plsc.subcore_barrier()

    slots = ((srcv0, dstv0, rows0, sem0, sems0),
             (srcv1, dstv1, rows1, sem1, sems1))

    def fire(slot, cid):
        srcv, dstv, rows, semg, _ = slots[slot]
        base = cid * C
        pltpu.sync_copy(src_hbm.at[pl.ds(base, C)], srcv)
        pltpu.sync_copy(dst_hbm.at[pl.ds(base, C)], dstv)
        pltpu.async_copy(x_hbm.at[srcv], rows, semg)

    def wait_scatters(slot):
        srcv, dstv, rows, _, sems = slots[slot]
        pltpu.make_async_copy(rows, agg_sh.at[dstv], sems).wait()
        pltpu.make_async_copy(onesv, cnt_sh.at[dstv], sems).wait()

    def drain_and_scatter(slot):
        srcv, dstv, rows, semg, sems = slots[slot]
        pltpu.make_async_copy(x_hbm.at[srcv], rows, semg).wait()
        pltpu.async_copy(rows, agg_sh.at[dstv], sems, add=True)
        pltpu.async_copy(onesv, cnt_sh.at[dstv], sems, add=True)

    fire(0, tid)

    @pl.loop(0, ITERS_A)
    def _(k):
        cid = k * NW + tid
        nxt = cid + NW

        @pl.when(cid < NCHUNK)
        def _():
            @pl.when(k % 2 == 0)
            def _():
                @pl.when(nxt < NCHUNK)
                def _():
                    @pl.when(k >= 1)
                    def _():
                        wait_scatters(1)
                    fire(1, nxt)
                drain_and_scatter(0)

            @pl.when(k % 2 == 1)
            def _():
                @pl.when(nxt < NCHUNK)
                def _():
                    wait_scatters(0)
                    fire(0, nxt)
                drain_and_scatter(1)

    wait_scatters(0)
    wait_scatters(1)

    plsc.subcore_barrier()

    for b in range(8):
        bid = b * 16 + s

        @pl.when(bid < NBLK)
        def _():
            pltpu.sync_copy(agg_sh.at[pl.ds(bid * ZBLK, ZBLK)],
                            rows0.at[pl.ds(0, ZBLK)])
            pltpu.sync_copy(rows0.at[pl.ds(0, ZBLK)],
                            agg_hbm.at[c].at[pl.ds(bid * ZBLK, ZBLK)])

    @pl.when(s < 5)
    def _():
        pltpu.sync_copy(cnt_sh.at[pl.ds(s * 2048, 2048)], cstage)
        pltpu.sync_copy(cstage, cnt_hbm.at[pl.ds(c * NP + s * 2048, 2048)])


@functools.partial(
    pl.kernel,
    mesh=_mesh,
    out_type=jax.ShapeDtypeStruct((2, N, D), jnp.float32),
    compiler_params=_sc_params,
    scratch_types=[
        pltpu.VMEM((CS,), jnp.int32),
        pltpu.VMEM((CS,), jnp.int32),
        pltpu.VMEM((160,), jnp.int32),
        pltpu.VMEM((160,), jnp.int32),
        pltpu.VMEM((B,), jnp.int32),
        pltpu.VMEM((B,), jnp.int32),
        pltpu.VMEM((B, D), jnp.float32),
        pltpu.VMEM((B,), jnp.int32),
        pltpu.VMEM((B,), jnp.int32),
        pltpu.VMEM((B, D), jnp.float32),
        pltpu.VMEM((B,), jnp.int32),
        pltpu.VMEM((B,), jnp.int32),
        pltpu.VMEM((B, D), jnp.float32),
        pltpu.VMEM((R2 + 1, D), jnp.float32),
        pltpu.SemaphoreType.DMA,
        pltpu.SemaphoreType.DMA,
        pltpu.SemaphoreType.DMA,
    ],
)
def _sc_seg_max(h_hbm, src_hbm, dst_hbm, z128_hbm, out_hbm,
                sbuf, dbuf, cdst, cloc,
                idx0, sav0, rows0, idx1, sav1, rows1, idx2, sav2, rows2,
                acc, sem0, sem1, sem2):
    c = lax.axis_index("c")
    s = lax.axis_index("s")
    lo = s * R2
    hi = lo + R2

    iota16 = lax.iota(jnp.int32, 16)

    for b in range(8):
        pltpu.sync_copy(z128_hbm, acc.at[pl.ds(b * ZBLK, ZBLK)])
    pltpu.sync_copy(z128_hbm.at[pl.ds(0, 1)], acc.at[pl.ds(R2, 1)])

    for v in range(10):
        cdst[pl.ds(v * 16, 16)] = iota16 + v * 16
        cloc[pl.ds(v * 16, 16)] = jnp.full((16,), R2, jnp.int32)

    slots = ((idx0, sav0, rows0, sem0), (idx1, sav1, rows1, sem1),
             (idx2, sav2, rows2, sem2))

    def fire(slot):
        idxb, sav, rows, sem = slots[slot]
        for v in range(B // 16):
            idxb[pl.ds(v * 16, 16)] = cdst[pl.ds(v * 16, 16)]
            sav[pl.ds(v * 16, 16)] = cloc[pl.ds(v * 16, 16)]
        pltpu.async_copy(h_hbm.at[idxb], rows, sem)

    def process(slot):
        idxb, sav, rows, sem = slots[slot]
        pltpu.make_async_copy(h_hbm.at[idxb], rows, sem).wait()

        @pl.loop(0, B // 16)
        def _(v):
            loc16 = sav[pl.ds(v * 16, 16)]
            for j2 in range(16):
                loc = loc16[j2]
                j = v * 16 + j2
                for g in range(D // 16):
                    a = acc[loc, pl.ds(g * 16, 16)]
                    r = rows[j, pl.ds(g * 16, 16)]
                    acc[loc, pl.ds(g * 16, 16)] = jnp.maximum(a, r)

    def chunk_body(kc, carry):
        base = c * EH + kc * CS
        pltpu.sync_copy(src_hbm.at[pl.ds(base, CS)], sbuf)
        pltpu.sync_copy(dst_hbm.at[pl.ds(base, CS)], dbuf)

        def group_body(g, carry):
            ptr, nf = carry
            sv = sbuf[pl.ds(g * 16, 16)]
            dv = dbuf[pl.ds(g * 16, 16)]
            m = (sv >= lo) & (sv < hi)
            plsc.store_compressed(cloc.at[pl.ds(ptr, 16)], sv - lo, mask=m)
            plsc.store_compressed(cdst.at[pl.ds(ptr, 16)], dv, mask=m)
            ptr = ptr + plsc.all_reduce_population_count(m)[0]

            @pl.when(ptr >= B)
            def _():
                for p in range(3):
                    @pl.when(nf % 3 == p)
                    def _(p=p):
                        fire(p)

                        @pl.when(nf > 1)
                        def _():
                            process((p + 1) % 3)

                cdst[pl.ds(0, 16)] = cdst[pl.ds(B, 16)]
                cloc[pl.ds(0, 16)] = cloc[pl.ds(B, 16)]

            hit = ptr >= B
            return (jnp.where(hit, ptr - B, ptr),
                    jnp.where(hit, nf + 1, nf))

        return lax.fori_loop(0, GROUPS, group_body, carry)

    ptr, nf = lax.fori_loop(0, NSCAN, chunk_body,
                            (jnp.int32(0), jnp.int32(0)))

    @pl.when(nf > 1)
    def _():
        for p in range(3):
            @pl.when(nf % 3 == p)
            def _(p=p):
                process((p + 1) % 3)

    @pl.when(nf > 0)
    def _():
        for p in range(3):
            @pl.when(nf % 3 == p)
            def _(p=p):
                process((p + 2) % 3)

    @pl.when(ptr > 0)
    def _():
        for v in range(B // 16):
            lv = cloc[pl.ds(v * 16, 16)]
            m = (iota16 + (v * 16)) < ptr
            cloc[pl.ds(v * 16, 16)] = jnp.where(m, lv, jnp.int32(R2))
        fire(0)
        process(0)

    for b in range(8):
        gid = s * 8 + b

        @pl.when(gid < NBLK)
        def _():
            pltpu.sync_copy(acc.at[pl.ds(b * ZBLK, ZBLK)],
                            out_hbm.at[c].at[pl.ds(s * R2 + b * ZBLK, ZBLK)])


def _tc_mlp1_body(a0, a1, c0, c1, w1, b1, h):
    agg = a0[...] + a1[...]
    cnt = c0[...] + c1[...]
    m = agg / jnp.maximum(cnt, 1.0)
    z = jnp.dot(m, w1[...], preferred_element_type=jnp.float32) + b1[...]
    h[...] = jnp.maximum(z, 0.0)


def _tc_mlp2_body(x, nm0, nm1, w2a, w2b, b2, o):
    nm = jnp.maximum(nm0[...], nm1[...])
    o[...] = (jnp.dot(x[...], w2a[...], preferred_element_type=jnp.float32)
              + jnp.dot(nm, w2b[...], preferred_element_type=jnp.float32)
              + b2[...])


_BR = 1024


def _mlp1(a0, a1, c0, c1, W1, b1):
    return pl.pallas_call(
        _tc_mlp1_body,
        grid=(NP // _BR,),
        in_specs=[
            pl.BlockSpec((_BR, D), lambda i: (i, 0)),
            pl.BlockSpec((_BR, D), lambda i: (i, 0)),
            pl.BlockSpec((_BR, 1), lambda i: (i, 0)),
            pl.BlockSpec((_BR, 1), lambda i: (i, 0)),
            pl.BlockSpec((D, D), lambda i: (0, 0)),
            pl.BlockSpec((1, D), lambda i: (0, 0)),
        ],
        out_specs=pl.BlockSpec((_BR, D), lambda i: (i, 0)),
        out_shape=jax.ShapeDtypeStruct((N, D), jnp.float32),
    )(a0, a1, c0, c1, W1, b1)


def _mlp2(x, nm0, nm1, W2a, W2b, b2):
    return pl.pallas_call(
        _tc_mlp2_body,
        grid=(NP // _BR,),
        in_specs=[
            pl.BlockSpec((_BR, D), lambda i: (i, 0)),
            pl.BlockSpec((_BR, D), lambda i: (i, 0)),
            pl.BlockSpec((_BR, D), lambda i: (i, 0)),
            pl.BlockSpec((D, D), lambda i: (0, 0)),
            pl.BlockSpec((D, D), lambda i: (0, 0)),
            pl.BlockSpec((1, D), lambda i: (0, 0)),
        ],
        out_specs=pl.BlockSpec((_BR, D), lambda i: (i, 0)),
        out_shape=jax.ShapeDtypeStruct((N, D), jnp.float32),
    )(x, nm0, nm1, W2a, W2b, b2)


def kernel(x, edge_index, W1, b1, W2, b2):
    src = edge_index[0].astype(jnp.int32)
    dst = edge_index[1].astype(jnp.int32)
    z128 = jnp.zeros((ZBLK, D), jnp.float32)
    z2k = jnp.zeros((2048,), jnp.float32)
    ones1 = jnp.ones((C,), jnp.float32)
    agg2, cnt2 = _sc_gather_add(x, src, dst, z128, z2k, ones1)
    cb = cnt2.reshape(2, NP, 1)
    hedge = _mlp1(agg2[0], agg2[1], cb[0], cb[1], W1, b1.reshape(1, D))
    nmax2 = _sc_seg_max(hedge, src, dst, z128)
    return _mlp2(x, nmax2[0], nmax2[1], W2[:D], W2[D:], b2.reshape(1, D))


# speedup vs baseline: 1.0553x; 1.0553x over previous
import functools

import jax
import jax.numpy as jnp
from jax import lax
from jax.experimental import pallas as pl
from jax.experimental.pallas import tpu as pltpu
from jax.experimental.pallas import tpu_sc as plsc

N = 10000
NP = 10240
E = 320000
D = 128

NW = 32
C = 128
NCHUNK = E // C
ITERS_A = (NCHUNK + NW - 1) // NW

ZBLK = 80
NBLK = N // ZBLK

R2 = 640
EH = E // 2
CS = 1600
NSCAN = EH // CS
GROUPS = CS // 16
B = 96

_mesh = plsc.VectorSubcoreMesh(core_axis_name="c", subcore_axis_name="s")
_sc_params = pltpu.CompilerParams(needs_layout_passes=False)


@functools.partial(
    pl.kernel,
    mesh=_mesh,
    out_type=[
        jax.ShapeDtypeStruct((2, N, D), jnp.float32),
        jax.ShapeDtypeStruct((2 * NP,), jnp.float32),
    ],
    compiler_params=_sc_params,
    scratch_types=[
        pltpu.VMEM((C,), jnp.int32),
        pltpu.VMEM((C,), jnp.int32),
        pltpu.VMEM((C, D), jnp.float32),
        pltpu.VMEM((C,), jnp.int32),
        pltpu.VMEM((C,), jnp.int32),
        pltpu.VMEM((C, D), jnp.float32),
        pltpu.VMEM((C,), jnp.float32),
        pltpu.VMEM((2048,), jnp.float32),
        pltpu.VMEM_SHARED((N, D), jnp.float32),
        pltpu.VMEM_SHARED((NP,), jnp.float32),
        pltpu.SemaphoreType.DMA,
        pltpu.SemaphoreType.DMA,
        pltpu.SemaphoreType.DMA,
        pltpu.SemaphoreType.DMA,
    ],
)
def _sc_gather_add(x_hbm, src_hbm, dst_hbm, z128_hbm, z2k_hbm, ones_hbm,
                   agg_hbm, cnt_hbm,
                   srcv0, dstv0, rows0, srcv1, dstv1, rows1,
                   onesv, cstage, agg_sh, cnt_sh, sem0, sem1, sems0, sems1):
    c = lax.axis_index("c")
    s = lax.axis_index("s")
    tid = s * 2 + c

    pltpu.sync_copy(z128_hbm, rows0.at[pl.ds(0, ZBLK)])
    pltpu.sync_copy(ones_hbm, onesv)
    pltpu.sync_copy(z2k_hbm, cstage)
    for b in range(8):
        bid = b * 16 + s

        @pl.when(bid < NBLK)
        def _():
            pltpu.sync_copy(rows0.at[pl.ds(0, ZBLK)],
                            agg_sh.at[pl.ds(bid * ZBLK, ZBLK)])

    @pl.when(s < 5)
    def _():
        pltpu.sync_copy(cstage, cnt_sh.at[pl.ds(s * 2048, 2048)])

    plsc.subcore_barrier()

    slots = ((srcv0, dstv0, rows0, sem0, sems0),
             (srcv1, dstv1, rows1, sem1, sems1))

    def fire(slot, cid):
        srcv, dstv, rows, semg, _ = slots[slot]
        base = cid * C
        pltpu.sync_copy(src_hbm.at[pl.ds(base, C)], srcv)
        pltpu.sync_copy(dst_hbm.at[pl.ds(base, C)], dstv)
        pltpu.async_copy(x_hbm.at[srcv], rows, semg)

    def wait_scatters(slot):
        srcv, dstv, rows, _, sems = slots[slot]
        pltpu.make_async_copy(rows, agg_sh.at[dstv], sems).wait()
        pltpu.make_async_copy(onesv, cnt_sh.at[dstv], sems).wait()

    def drain_and_scatter(slot):
        srcv, dstv, rows, semg, sems = slots[slot]
        pltpu.make_async_copy(x_hbm.at[srcv], rows, semg).wait()
        pltpu.async_copy(rows, agg_sh.at[dstv], sems, add=True)
        pltpu.async_copy(onesv, cnt_sh.at[dstv], sems, add=True)

    fire(0, tid)

    @pl.loop(0, ITERS_A)
    def _(k):
        cid = k * NW + tid
        nxt = cid + NW

        @pl.when(cid < NCHUNK)
        def _():
            @pl.when(k % 2 == 0)
            def _():
                @pl.when(nxt < NCHUNK)
                def _():
                    @pl.when(k >= 1)
                    def _():
                        wait_scatters(1)
                    fire(1, nxt)
                drain_and_scatter(0)

            @pl.when(k % 2 == 1)
            def _():
                @pl.when(nxt < NCHUNK)
                def _():
                    wait_scatters(0)
                    fire(0, nxt)
                drain_and_scatter(1)

    wait_scatters(0)
    wait_scatters(1)

    plsc.subcore_barrier()

    for b in range(8):
        bid = b * 16 + s

        @pl.when(bid < NBLK)
        def _():
            pltpu.sync_copy(agg_sh.at[pl.ds(bid * ZBLK, ZBLK)],
                            rows0.at[pl.ds(0, ZBLK)])
            pltpu.sync_copy(rows0.at[pl.ds(0, ZBLK)],
                            agg_hbm.at[c].at[pl.ds(bid * ZBLK, ZBLK)])

    @pl.when(s < 5)
    def _():
        pltpu.sync_copy(cnt_sh.at[pl.ds(s * 2048, 2048)], cstage)
        pltpu.sync_copy(cstage, cnt_hbm.at[pl.ds(c * NP + s * 2048, 2048)])


@functools.partial(
    pl.kernel,
    mesh=_mesh,
    out_type=jax.ShapeDtypeStruct((2, N, D), jnp.float32),
    compiler_params=_sc_params,
    scratch_types=[
        pltpu.VMEM((CS,), jnp.int32),
        pltpu.VMEM((CS,), jnp.int32),
        pltpu.VMEM((160,), jnp.int32),
        pltpu.VMEM((160,), jnp.int32),
        pltpu.VMEM((B,), jnp.int32),
        pltpu.VMEM((B,), jnp.int32),
        pltpu.VMEM((B, D), jnp.float32),
        pltpu.VMEM((B,), jnp.int32),
        pltpu.VMEM((B,), jnp.int32),
        pltpu.VMEM((B, D), jnp.float32),
        pltpu.VMEM((B,), jnp.int32),
        pltpu.VMEM((B,), jnp.int32),
        pltpu.VMEM((B, D), jnp.float32),
        pltpu.VMEM((R2 + 1, D), jnp.float32),
        pltpu.SemaphoreType.DMA,
        pltpu.SemaphoreType.DMA,
        pltpu.SemaphoreType.DMA,
    ],
)
def _sc_seg_max(h_hbm, src_hbm, dst_hbm, z128_hbm, out_hbm,
                sbuf, dbuf, cdst, cloc,
                idx0, sav0, rows0, idx1, sav1, rows1, idx2, sav2, rows2,
                acc, sem0, sem1, sem2):
    c = lax.axis_index("c")
    s = lax.axis_index("s")
    lo = s * R2
    hi = lo + R2

    iota16 = lax.iota(jnp.int32, 16)

    for b in range(8):
        pltpu.sync_copy(z128_hbm, acc.at[pl.ds(b * ZBLK, ZBLK)])
    pltpu.sync_copy(z128_hbm.at[pl.ds(0, 1)], acc.at[pl.ds(R2, 1)])

    for v in range(10):
        cdst[pl.ds(v * 16, 16)] = iota16 + v * 16
        cloc[pl.ds(v * 16, 16)] = jnp.full((16,), R2, jnp.int32)

    slots = ((idx0, sav0, rows0, sem0), (idx1, sav1, rows1, sem1),
             (idx2, sav2, rows2, sem2))

    def fire(slot):
        idxb, sav, rows, sem = slots[slot]
        for v in range(B // 16):
            idxb[pl.ds(v * 16, 16)] = cdst[pl.ds(v * 16, 16)]
            sav[pl.ds(v * 16, 16)] = cloc[pl.ds(v * 16, 16)]
        pltpu.async_copy(h_hbm.at[idxb], rows, sem)

    def process(slot):
        idxb, sav, rows, sem = slots[slot]
        pltpu.make_async_copy(h_hbm.at[idxb], rows, sem).wait()

        @pl.loop(0, B // 16)
        def _(v):
            loc16 = sav[pl.ds(v * 16, 16)]
            for j2 in range(16):
                loc = loc16[j2]
                j = v * 16 + j2
                for g in range(D // 16):
                    a = acc[loc, pl.ds(g * 16, 16)]
                    r = rows[j, pl.ds(g * 16, 16)]
                    acc[loc, pl.ds(g * 16, 16)] = jnp.maximum(a, r)

    def chunk_body(kc, carry):
        base = c * EH + kc * CS
        pltpu.sync_copy(src_hbm.at[pl.ds(base, CS)], sbuf)
        pltpu.sync_copy(dst_hbm.at[pl.ds(base, CS)], dbuf)

        def pair_body(g2, carry):
            ptr, nf = carry
            for h in range(2):
                g16 = g2 * 32 + h * 16
                sv = sbuf[pl.ds(g16, 16)]
                dv = dbuf[pl.ds(g16, 16)]
                lv = sv - lo
                m = plsc.bitcast(lv, jnp.uint32) < jnp.uint32(R2)
                plsc.store_compressed(cloc.at[pl.ds(ptr, 16)], lv, mask=m)
                plsc.store_compressed(cdst.at[pl.ds(ptr, 16)], dv, mask=m)
                ptr = ptr + plsc.all_reduce_population_count(m)[0]

            @pl.when(ptr >= B)
            def _():
                for p in range(3):
                    @pl.when(nf % 3 == p)
                    def _(p=p):
                        fire(p)

                        @pl.when(nf > 1)
                        def _():
                            process((p + 1) % 3)

                cdst[pl.ds(0, 16)] = cdst[pl.ds(B, 16)]
                cdst[pl.ds(16, 16)] = cdst[pl.ds(B + 16, 16)]
                cloc[pl.ds(0, 16)] = cloc[pl.ds(B, 16)]
                cloc[pl.ds(16, 16)] = cloc[pl.ds(B + 16, 16)]

            hit = ptr >= B
            return (jnp.where(hit, ptr - B, ptr),
                    jnp.where(hit, nf + 1, nf))

        return lax.fori_loop(0, GROUPS // 2, pair_body, carry)

    ptr, nf = lax.fori_loop(0, NSCAN, chunk_body,
                            (jnp.int32(0), jnp.int32(0)))

    @pl.when(nf > 1)
    def _():
        for p in range(3):
            @pl.when(nf % 3 == p)
            def _(p=p):
                process((p + 1) % 3)

    @pl.when(nf > 0)
    def _():
        for p in range(3):
            @pl.when(nf % 3 == p)
            def _(p=p):
                process((p + 2) % 3)

    @pl.when(ptr > 0)
    def _():
        for v in range(B // 16):
            lv = cloc[pl.ds(v * 16, 16)]
            m = (iota16 + (v * 16)) < ptr
            cloc[pl.ds(v * 16, 16)] = jnp.where(m, lv, jnp.int32(R2))
        fire(0)
        process(0)

    for b in range(8):
        gid = s * 8 + b

        @pl.when(gid < NBLK)
        def _():
            pltpu.sync_copy(acc.at[pl.ds(b * ZBLK, ZBLK)],
                            out_hbm.at[c].at[pl.ds(s * R2 + b * ZBLK, ZBLK)])


def _tc_mlp1_body(a0, a1, c0, c1, w1, b1, h):
    agg = a0[...] + a1[...]
    cnt = c0[...] + c1[...]
    m = agg / jnp.maximum(cnt, 1.0)
    z = jnp.dot(m, w1[...], preferred_element_type=jnp.float32) + b1[...]
    h[...] = jnp.maximum(z, 0.0)


def _tc_mlp2_body(x, nm0, nm1, w2a, w2b, b2, o):
    nm = jnp.maximum(nm0[...], nm1[...])
    o[...] = (jnp.dot(x[...], w2a[...], preferred_element_type=jnp.float32)
              + jnp.dot(nm, w2b[...], preferred_element_type=jnp.float32)
              + b2[...])


_BR = 1024


def _mlp1(a0, a1, c0, c1, W1, b1):
    return pl.pallas_call(
        _tc_mlp1_body,
        grid=(NP // _BR,),
        in_specs=[
            pl.BlockSpec((_BR, D), lambda i: (i, 0)),
            pl.BlockSpec((_BR, D), lambda i: (i, 0)),
            pl.BlockSpec((_BR, 1), lambda i: (i, 0)),
            pl.BlockSpec((_BR, 1), lambda i: (i, 0)),
            pl.BlockSpec((D, D), lambda i: (0, 0)),
            pl.BlockSpec((1, D), lambda i: (0, 0)),
        ],
        out_specs=pl.BlockSpec((_BR, D), lambda i: (i, 0)),
        out_shape=jax.ShapeDtypeStruct((N, D), jnp.float32),
    )(a0, a1, c0, c1, W1, b1)


def _mlp2(x, nm0, nm1, W2a, W2b, b2):
    return pl.pallas_call(
        _tc_mlp2_body,
        grid=(NP // _BR,),
        in_specs=[
            pl.BlockSpec((_BR, D), lambda i: (i, 0)),
            pl.BlockSpec((_BR, D), lambda i: (i, 0)),
            pl.BlockSpec((_BR, D), lambda i: (i, 0)),
            pl.BlockSpec((D, D), lambda i: (0, 0)),
            pl.BlockSpec((D, D), lambda i: (0, 0)),
            pl.BlockSpec((1, D), lambda i: (0, 0)),
        ],
        out_specs=pl.BlockSpec((_BR, D), lambda i: (i, 0)),
        out_shape=jax.ShapeDtypeStruct((N, D), jnp.float32),
    )(x, nm0, nm1, W2a, W2b, b2)


def kernel(x, edge_index, W1, b1, W2, b2):
    src = edge_index[0].astype(jnp.int32)
    dst = edge_index[1].astype(jnp.int32)
    z128 = jnp.zeros((ZBLK, D), jnp.float32)
    z2k = jnp.zeros((2048,), jnp.float32)
    ones1 = jnp.ones((C,), jnp.float32)
    agg2, cnt2 = _sc_gather_add(x, src, dst, z128, z2k, ones1)
    cb = cnt2.reshape(2, NP, 1)
    hedge = _mlp1(agg2[0], agg2[1], cb[0], cb[1], W1, b1.reshape(1, D))
    nmax2 = _sc_seg_max(hedge, src, dst, z128)
    return _mlp2(x, nmax2[0], nmax2[1], W2[:D], W2[D:], b2.reshape(1, D))


# speedup vs baseline: 1.0930x; 1.0357x over previous
import functools

import jax
import jax.numpy as jnp
from jax import lax
from jax.experimental import pallas as pl
from jax.experimental.pallas import tpu as pltpu
from jax.experimental.pallas import tpu_sc as plsc

N = 10000
NP = 10240
E = 320000
D = 128

NW = 32
C = 128
NCHUNK = E // C
ITERS_A = (NCHUNK + NW - 1) // NW

ZBLK = 80
NBLK = N // ZBLK

R2 = 640
EH = E // 2
CS = 1600
NSCAN = EH // CS
GROUPS = CS // 16
B = 96

_mesh = plsc.VectorSubcoreMesh(core_axis_name="c", subcore_axis_name="s")
_sc_params = pltpu.CompilerParams(needs_layout_passes=False)


@functools.partial(
    pl.kernel,
    mesh=_mesh,
    out_type=[
        jax.ShapeDtypeStruct((2, N, D), jnp.float32),
        jax.ShapeDtypeStruct((2 * NP,), jnp.float32),
    ],
    compiler_params=_sc_params,
    scratch_types=[
        pltpu.VMEM((C,), jnp.int32),
        pltpu.VMEM((C,), jnp.int32),
        pltpu.VMEM((C, D), jnp.float32),
        pltpu.VMEM((C,), jnp.int32),
        pltpu.VMEM((C,), jnp.int32),
        pltpu.VMEM((C, D), jnp.float32),
        pltpu.VMEM((C,), jnp.float32),
        pltpu.VMEM((2048,), jnp.float32),
        pltpu.VMEM_SHARED((N, D), jnp.float32),
        pltpu.VMEM_SHARED((NP,), jnp.float32),
        pltpu.SemaphoreType.DMA,
        pltpu.SemaphoreType.DMA,
        pltpu.SemaphoreType.DMA,
        pltpu.SemaphoreType.DMA,
    ],
)
def _sc_gather_add(x_hbm, src_hbm, dst_hbm, z128_hbm, z2k_hbm, ones_hbm,
                   agg_hbm, cnt_hbm,
                   srcv0, dstv0, rows0, srcv1, dstv1, rows1,
                   onesv, cstage, agg_sh, cnt_sh, sem0, sem1, sems0, sems1):
    c = lax.axis_index("c")
    s = lax.axis_index("s")
    tid = s * 2 + c

    pltpu.sync_copy(z128_hbm, rows0.at[pl.ds(0, ZBLK)])
    pltpu.sync_copy(ones_hbm, onesv)
    pltpu.sync_copy(z2k_hbm, cstage)
    for b in range(8):
        bid = b * 16 + s

        @pl.when(bid < NBLK)
        def _():
            pltpu.sync_copy(rows0.at[pl.ds(0, ZBLK)],
                            agg_sh.at[pl.ds(bid * ZBLK, ZBLK)])

    @pl.when(s < 5)
    def _():
        pltpu.sync_copy(cstage, cnt_sh.at[pl.ds(s * 2048, 2048)])

    plsc.subcore_barrier()

    slots = ((srcv0, dstv0, rows0, sem0, sems0),
             (srcv1, dstv1, rows1, sem1, sems1))

    def fire(slot, cid):
        srcv, dstv, rows, semg, _ = slots[slot]
        base = cid * C
        pltpu.sync_copy(src_hbm.at[pl.ds(base, C)], srcv)
        pltpu.sync_copy(dst_hbm.at[pl.ds(base, C)], dstv)
        pltpu.async_copy(x_hbm.at[srcv], rows, semg)

    def wait_scatters(slot):
        srcv, dstv, rows, _, sems = slots[slot]
        pltpu.make_async_copy(rows, agg_sh.at[dstv], sems).wait()
        pltpu.make_async_copy(onesv, cnt_sh.at[dstv], sems).wait()

    def drain_and_scatter(slot):
        srcv, dstv, rows, semg, sems = slots[slot]
        pltpu.make_async_copy(x_hbm.at[srcv], rows, semg).wait()
        pltpu.async_copy(rows, agg_sh.at[dstv], sems, add=True)
        pltpu.async_copy(onesv, cnt_sh.at[dstv], sems, add=True)

    fire(0, tid)

    @pl.loop(0, ITERS_A)
    def _(k):
        cid = k * NW + tid
        nxt = cid + NW

        @pl.when(cid < NCHUNK)
        def _():
            @pl.when(k % 2 == 0)
            def _():
                @pl.when(nxt < NCHUNK)
                def _():
                    @pl.when(k >= 1)
                    def _():
                        wait_scatters(1)
                    fire(1, nxt)
                drain_and_scatter(0)

            @pl.when(k % 2 == 1)
            def _():
                @pl.when(nxt < NCHUNK)
                def _():
                    wait_scatters(0)
                    fire(0, nxt)
                drain_and_scatter(1)

    wait_scatters(0)
    wait_scatters(1)

    plsc.subcore_barrier()

    for b in range(8):
        bid = b * 16 + s

        @pl.when(bid < NBLK)
        def _():
            pltpu.sync_copy(agg_sh.at[pl.ds(bid * ZBLK, ZBLK)],
                            rows0.at[pl.ds(0, ZBLK)])
            pltpu.sync_copy(rows0.at[pl.ds(0, ZBLK)],
                            agg_hbm.at[c].at[pl.ds(bid * ZBLK, ZBLK)])

    @pl.when(s < 5)
    def _():
        pltpu.sync_copy(cnt_sh.at[pl.ds(s * 2048, 2048)], cstage)
        pltpu.sync_copy(cstage, cnt_hbm.at[pl.ds(c * NP + s * 2048, 2048)])


@functools.partial(
    pl.kernel,
    mesh=_mesh,
    out_type=jax.ShapeDtypeStruct((2, N, D), jnp.float32),
    compiler_params=_sc_params,
    scratch_types=[
        pltpu.VMEM((CS,), jnp.int32),
        pltpu.VMEM((CS,), jnp.int32),
        pltpu.VMEM((160,), jnp.int32),
        pltpu.VMEM((160,), jnp.int32),
        pltpu.VMEM((B,), jnp.int32),
        pltpu.VMEM((B,), jnp.int32),
        pltpu.VMEM((B, D), jnp.float32),
        pltpu.VMEM((B,), jnp.int32),
        pltpu.VMEM((B,), jnp.int32),
        pltpu.VMEM((B, D), jnp.float32),
        pltpu.VMEM((B,), jnp.int32),
        pltpu.VMEM((B,), jnp.int32),
        pltpu.VMEM((B, D), jnp.float32),
        pltpu.VMEM((R2 + 1, D), jnp.float32),
        pltpu.SemaphoreType.DMA,
        pltpu.SemaphoreType.DMA,
        pltpu.SemaphoreType.DMA,
    ],
)
def _sc_seg_max(h_hbm, src_hbm, dst_hbm, z128_hbm, out_hbm,
                sbuf, dbuf, cdst, cloc,
                idx0, sav0, rows0, idx1, sav1, rows1, idx2, sav2, rows2,
                acc, sem0, sem1, sem2):
    c = lax.axis_index("c")
    s = lax.axis_index("s")
    lo = s * R2
    hi = lo + R2

    iota16 = lax.iota(jnp.int32, 16)

    for b in range(8):
        pltpu.sync_copy(z128_hbm, acc.at[pl.ds(b * ZBLK, ZBLK)])
    pltpu.sync_copy(z128_hbm.at[pl.ds(0, 1)], acc.at[pl.ds(R2, 1)])

    for v in range(10):
        cdst[pl.ds(v * 16, 16)] = iota16 + v * 16
        cloc[pl.ds(v * 16, 16)] = jnp.full((16,), R2, jnp.int32)

    slots = ((idx0, sav0, rows0, sem0), (idx1, sav1, rows1, sem1),
             (idx2, sav2, rows2, sem2))

    def fire(slot):
        idxb, sav, rows, sem = slots[slot]
        for v in range(B // 16):
            idxb[pl.ds(v * 16, 16)] = cdst[pl.ds(v * 16, 16)]
            sav[pl.ds(v * 16, 16)] = cloc[pl.ds(v * 16, 16)]
        pltpu.async_copy(h_hbm.at[idxb], rows, sem)

    def process(slot):
        idxb, sav, rows, sem = slots[slot]
        pltpu.make_async_copy(h_hbm.at[idxb], rows, sem).wait()

        @pl.loop(0, B // 16)
        def _(v):
            loc16 = sav[pl.ds(v * 16, 16)]
            for j2 in range(16):
                loc = loc16[j2]
                j = v * 16 + j2
                for g in range(D // 16):
                    a = acc[loc, pl.ds(g * 16, 16)]
                    r = rows[j, pl.ds(g * 16, 16)]
                    acc[loc, pl.ds(g * 16, 16)] = jnp.maximum(a, r)

    def chunk_body(kc, carry):
        base = c * EH + kc * CS
        pltpu.sync_copy(src_hbm.at[pl.ds(base, CS)], sbuf)
        pltpu.sync_copy(dst_hbm.at[pl.ds(base, CS)], dbuf)

        def pair_body(g2, carry):
            ptr, nf = carry
            for h in range(4):
                g16 = g2 * 64 + h * 16
                sv = sbuf[pl.ds(g16, 16)]
                dv = dbuf[pl.ds(g16, 16)]
                lv = sv - lo
                m = plsc.bitcast(lv, jnp.uint32) < jnp.uint32(R2)
                plsc.store_compressed(cloc.at[pl.ds(ptr, 16)], lv, mask=m)
                plsc.store_compressed(cdst.at[pl.ds(ptr, 16)], dv, mask=m)
                ptr = ptr + plsc.all_reduce_population_count(m)[0]

            @pl.when(ptr >= B)
            def _():
                for p in range(3):
                    @pl.when(nf % 3 == p)
                    def _(p=p):
                        fire(p)

                        @pl.when(nf > 1)
                        def _():
                            process((p + 1) % 3)

                for q in range(4):
                    cdst[pl.ds(q * 16, 16)] = cdst[pl.ds(B + q * 16, 16)]
                    cloc[pl.ds(q * 16, 16)] = cloc[pl.ds(B + q * 16, 16)]

            hit = ptr >= B
            return (jnp.where(hit, ptr - B, ptr),
                    jnp.where(hit, nf + 1, nf))

        return lax.fori_loop(0, GROUPS // 4, pair_body, carry)

    ptr, nf = lax.fori_loop(0, NSCAN, chunk_body,
                            (jnp.int32(0), jnp.int32(0)))

    @pl.when(nf > 1)
    def _():
        for p in range(3):
            @pl.when(nf % 3 == p)
            def _(p=p):
                process((p + 1) % 3)

    @pl.when(nf > 0)
    def _():
        for p in range(3):
            @pl.when(nf % 3 == p)
            def _(p=p):
                process((p + 2) % 3)

    @pl.when(ptr > 0)
    def _():
        for v in range(B // 16):
            lv = cloc[pl.ds(v * 16, 16)]
            m = (iota16 + (v * 16)) < ptr
            cloc[pl.ds(v * 16, 16)] = jnp.where(m, lv, jnp.int32(R2))
        fire(0)
        process(0)

    for b in range(8):
        gid = s * 8 + b

        @pl.when(gid < NBLK)
        def _():
            pltpu.sync_copy(acc.at[pl.ds(b * ZBLK, ZBLK)],
                            out_hbm.at[c].at[pl.ds(s * R2 + b * ZBLK, ZBLK)])


def _tc_mlp1_body(a0, a1, c0, c1, w1, b1, h):
    agg = a0[...] + a1[...]
    cnt = c0[...] + c1[...]
    m = agg / jnp.maximum(cnt, 1.0)
    z = jnp.dot(m, w1[...], preferred_element_type=jnp.float32) + b1[...]
    h[...] = jnp.maximum(z, 0.0)


def _tc_mlp2_body(x, nm0, nm1, w2a, w2b, b2, o):
    nm = jnp.maximum(nm0[...], nm1[...])
    o[...] = (jnp.dot(x[...], w2a[...], preferred_element_type=jnp.float32)
              + jnp.dot(nm, w2b[...], preferred_element_type=jnp.float32)
              + b2[...])


_BR = 1024


def _mlp1(a0, a1, c0, c1, W1, b1):
    return pl.pallas_call(
        _tc_mlp1_body,
        grid=(NP // _BR,),
        in_specs=[
            pl.BlockSpec((_BR, D), lambda i: (i, 0)),
            pl.BlockSpec((_BR, D), lambda i: (i, 0)),
            pl.BlockSpec((_BR, 1), lambda i: (i, 0)),
            pl.BlockSpec((_BR, 1), lambda i: (i, 0)),
            pl.BlockSpec((D, D), lambda i: (0, 0)),
            pl.BlockSpec((1, D), lambda i: (0, 0)),
        ],
        out_specs=pl.BlockSpec((_BR, D), lambda i: (i, 0)),
        out_shape=jax.ShapeDtypeStruct((N, D), jnp.float32),
    )(a0, a1, c0, c1, W1, b1)


def _mlp2(x, nm0, nm1, W2a, W2b, b2):
    return pl.pallas_call(
        _tc_mlp2_body,
        grid=(NP // _BR,),
        in_specs=[
            pl.BlockSpec((_BR, D), lambda i: (i, 0)),
            pl.BlockSpec((_BR, D), lambda i: (i, 0)),
            pl.BlockSpec((_BR, D), lambda i: (i, 0)),
            pl.BlockSpec((D, D), lambda i: (0, 0)),
            pl.BlockSpec((D, D), lambda i: (0, 0)),
            pl.BlockSpec((1, D), lambda i: (0, 0)),
        ],
        out_specs=pl.BlockSpec((_BR, D), lambda i: (i, 0)),
        out_shape=jax.ShapeDtypeStruct((N, D), jnp.float32),
    )(x, nm0, nm1, W2a, W2b, b2)


def kernel(x, edge_index, W1, b1, W2, b2):
    src = edge_index[0].astype(jnp.int32)
    dst = edge_index[1].astype(jnp.int32)
    z128 = jnp.zeros((ZBLK, D), jnp.float32)
    z2k = jnp.zeros((2048,), jnp.float32)
    ones1 = jnp.ones((C,), jnp.float32)
    agg2, cnt2 = _sc_gather_add(x, src, dst, z128, z2k, ones1)
    cb = cnt2.reshape(2, NP, 1)
    hedge = _mlp1(agg2[0], agg2[1], cb[0], cb[1], W1, b1.reshape(1, D))
    nmax2 = _sc_seg_max(hedge, src, dst, z128)
    return _mlp2(x, nmax2[0], nmax2[1], W2[:D], W2[D:], b2.reshape(1, D))
